# Initial kernel scaffold; baseline (speedup 1.0000x reference)
import jax
import jax.numpy as jnp
from jax.experimental import pallas as pl


def kernel(z, Wg, bg, ln_w, ln_b, W1, b1, W2, b2):
    raise NotImplementedError("write your pallas kernel here")


# speedup vs baseline: 4.3442x; 4.3442x over previous
import functools

import jax
import jax.numpy as jnp
from jax import lax
from jax.experimental import pallas as pl
from jax.experimental.pallas import tpu as pltpu
from jax.experimental.pallas import tpu_sc as plsc

_B, _D, _H, _E = 2048, 1024, 4096, 8
_T = 128
_NB = _B // _T + _E
_P = _NB * _T
_HB = 1024
_BT = 256
_EPS = 1e-5

_info = plsc.get_sparse_core_info()
_NC, _NS = _info.num_cores, _info.num_subcores
_NW = _NC * _NS
_CB = _B // _NW


def _gate_body(z_ref, wg_ref, bg_ref,
               g_ref, idx_ref, rank_ref, counts_ref, ln_ref, runc_ref):
    i = pl.program_id(0)
    zb = z_ref[...]
    logits = lax.dot_general(zb, wg_ref[...], (((1,), (1,)), ((), ())),
                             preferred_element_type=jnp.float32)
    logits = logits + bg_ref[...]
    m = jnp.max(logits, axis=1, keepdims=True)
    p = jnp.exp(logits - m)
    g_ref[...] = p / jnp.sum(p, axis=1, keepdims=True)

    iota_e = lax.broadcasted_iota(jnp.int32, (_BT, _E), 1)
    idx2 = jnp.min(jnp.where(logits == m, iota_e, _E), axis=1,
                   keepdims=True)
    idx_ref[...] = idx2

    onehot = (iota_e == idx2).astype(jnp.float32)
    ti = lax.broadcasted_iota(jnp.int32, (_BT, _BT), 0)
    tj = lax.broadcasted_iota(jnp.int32, (_BT, _BT), 1)
    tri = (ti > tj).astype(jnp.float32)
    prev = lax.dot_general(tri, onehot, (((1,), (0,)), ((), ())),
                           preferred_element_type=jnp.float32)
    runc = jnp.where(i == 0, jnp.zeros((1, _E), jnp.float32), runc_ref[...])
    rank = jnp.sum((prev + runc) * onehot, axis=1, keepdims=True)
    rank_ref[...] = rank.astype(jnp.int32)
    newc = runc + jnp.sum(onehot, axis=0, keepdims=True)
    runc_ref[...] = newc
    counts_ref[...] = newc.astype(jnp.int32)

    mu = jnp.mean(zb, axis=1, keepdims=True)
    var = jnp.mean((zb - mu) ** 2, axis=1, keepdims=True)
    ln_ref[...] = (zb - mu) / jnp.sqrt(var + _EPS)


_gate_call = pl.pallas_call(
    _gate_body,
    grid=(_B // _BT,),
    in_specs=[
        pl.BlockSpec((_BT, _D), lambda i: (i, 0)),
        pl.BlockSpec((_E, _D), lambda i: (0, 0)),
        pl.BlockSpec((1, _E), lambda i: (0, 0)),
    ],
    out_specs=[
        pl.BlockSpec((_BT, _E), lambda i: (i, 0)),
        pl.BlockSpec((_BT, 1), lambda i: (i, 0)),
        pl.BlockSpec((_BT, 1), lambda i: (i, 0)),
        pl.BlockSpec((1, _E), lambda i: (0, 0)),
        pl.BlockSpec((_BT, _D), lambda i: (i, 0)),
    ],
    out_shape=[
        jax.ShapeDtypeStruct((_B, _E), jnp.float32),
        jax.ShapeDtypeStruct((_B, 1), jnp.int32),
        jax.ShapeDtypeStruct((_B, 1), jnp.int32),
        jax.ShapeDtypeStruct((1, _E), jnp.int32),
        jax.ShapeDtypeStruct((_B, _D), jnp.float32),
    ],
    scratch_shapes=[pltpu.VMEM((1, _E), jnp.float32)],
)


_sc_mesh = plsc.VectorSubcoreMesh(core_axis_name="c", subcore_axis_name="s")


@functools.partial(
    pl.kernel, mesh=_sc_mesh,
    out_type=jax.ShapeDtypeStruct((_P, _D), jnp.float32),
    scratch_types=[pltpu.VMEM((_CB,), jnp.int32),
                   pltpu.VMEM((_CB, _D), jnp.float32),
                   pltpu.SemaphoreType.DMA],
)
def _dispatch(ln_hbm, slots_hbm, xpad_hbm, slot_v, rows_v, sem):
    wid = lax.axis_index("s") * _NC + lax.axis_index("c")
    base = wid * _CB
    pltpu.sync_copy(slots_hbm.at[pl.ds(base, _CB)], slot_v)
    pltpu.sync_copy(ln_hbm.at[pl.ds(base, _CB)], rows_v)
    pltpu.async_copy(rows_v, xpad_hbm.at[slot_v], sem).wait()


def _ffn_body(be_ref, x_ref, w1_ref, b1_ref, w2_ref, lnw_ref, lnb_ref,
              b2_ref, out_ref):
    j = pl.program_id(0)
    xs = x_ref[0] * lnw_ref[0] + lnb_ref[0]
    h = lax.dot_general(xs, w1_ref[0], (((1,), (1,)), ((), ())),
                        preferred_element_type=jnp.float32)
    h = h + b1_ref[0]
    h = 0.5 * h * (1.0 + lax.erf(h * 0.7071067811865476))
    part = lax.dot_general(w2_ref[0], h, (((1,), (1,)), ((), ())),
                           preferred_element_type=jnp.float32)
    out_ref[...] = (part + jnp.where(j == 0, b2_ref[0], 0.0))[:, None, :]


_ffn_call = pl.pallas_call(
    _ffn_body,
    grid_spec=pltpu.PrefetchScalarGridSpec(
        num_scalar_prefetch=1,
        grid=(_H // _HB, _NB),
        in_specs=[
            pl.BlockSpec((1, _T, _D), lambda j, i, be: (i, 0, 0)),
            pl.BlockSpec((1, _HB, _D), lambda j, i, be: (be[i], j, 0)),
            pl.BlockSpec((1, 1, _HB),
                         lambda j, i, be: (be[i] * (_H // _HB) + j, 0, 0)),
            pl.BlockSpec((1, 1, _HB),
                         lambda j, i, be: (be[i] * (_H // _HB) + j, 0, 0)),
            pl.BlockSpec((1, 1, _D), lambda j, i, be: (be[i], 0, 0)),
            pl.BlockSpec((1, 1, _D), lambda j, i, be: (be[i], 0, 0)),
            pl.BlockSpec((1, 1, 1), lambda j, i, be: (be[i], 0, 0)),
        ],
        out_specs=pl.BlockSpec((1, 1, _T),
                               lambda j, i, be: (j * _NB + i, 0, 0)),
    ),
    out_shape=jax.ShapeDtypeStruct(((_H // _HB) * _NB, 1, _T), jnp.float32),
)


@functools.partial(
    pl.kernel, mesh=_sc_mesh,
    out_type=jax.ShapeDtypeStruct((_B,), jnp.float32),
    scratch_types=[pltpu.VMEM((_CB,), jnp.int32),
                   pltpu.VMEM((_CB,), jnp.float32),
                   pltpu.SemaphoreType.DMA],
)
def _combine(ypad_hbm, slots_hbm, out_hbm, slot_v, val_v, sem):
    wid = lax.axis_index("s") * _NC + lax.axis_index("c")
    base = wid * _CB
    pltpu.sync_copy(slots_hbm.at[pl.ds(base, _CB)], slot_v)
    pltpu.async_copy(ypad_hbm.at[slot_v], val_v, sem).wait()
    pltpu.sync_copy(val_v, out_hbm.at[pl.ds(base, _CB)])


def kernel(z, Wg, bg, ln_w, ln_b, W1, b1, W2, b2):
    g, idx2, rank2, counts2, lncore = _gate_call(z, Wg, bg.reshape(1, _E))
    idx = idx2.reshape(_B)
    rank = rank2.reshape(_B)
    counts = counts2.reshape(_E)

    nblk = (counts + _T - 1) // _T
    cum = jnp.cumsum(nblk)
    pad_off = ((cum - nblk) * _T).astype(jnp.int32)
    ks = jnp.arange(_NB, dtype=jnp.int32)
    block_expert = jnp.minimum(
        jnp.sum((ks[:, None] >= cum[None, :]).astype(jnp.int32), axis=1),
        _E - 1).astype(jnp.int32)
    slots = rank + jnp.sum(
        jnp.where(idx[:, None] == jnp.arange(_E, dtype=jnp.int32)[None, :],
                  pad_off[None, :], 0), axis=1).astype(jnp.int32)

    xpad = _dispatch(lncore, slots)
    partials = _ffn_call(block_expert, xpad.reshape(_NB, _T, _D),
                         W1, b1.reshape(_E * (_H // _HB), 1, _HB),
                         W2.reshape(_E * (_H // _HB), 1, _HB),
                         ln_w.reshape(_E, 1, _D), ln_b.reshape(_E, 1, _D),
                         b2.reshape(_E, 1, 1))
    ypad = jnp.sum(partials.reshape(_H // _HB, _NB, _T), axis=0).reshape(_P)
    logits = _combine(ypad, slots)
    return (logits, g)
